# baseline (speedup 1.0000x reference)
import jax
import jax.numpy as jnp
from jax import lax
from jax.experimental import pallas as pl
from jax.experimental.pallas import tpu as pltpu
from jax.experimental.pallas import tpu_sc as plsc

_B, _C, _E, _P = 32, 64, 64, 256
_PH = _P // 2
_L = 16
_NC, _NS = 2, 16
_NW = _NC * _NS
_CPW = _C // _NW
_BT = 16
_NSLAB = (_B - _BT) * _CPW


def _gather_slab(in_ref, out_ref, idx_v, cl):
    base = [idx_v[pl.ds(cl * _P + j * _L, _L)] for j in range(_P // _L)]

    @plsc.parallel_loop(0, _E, unroll=2)
    def e_body(e):
        e_vec = jnp.full((_L,), e, dtype=jnp.int32)
        for j in range(_P // _L):
            g = plsc.load_gather(in_ref, [e_vec, base[j]])
            out_ref[e, pl.ds(j * _L, _L)] = g


def _shuffle_body(x_hbm, idx_hbm, out_hbm, idx_v, in0, in1, out0, out1,
                  si0, si1, so0, so1):
    wid = lax.axis_index("s") * _NC + lax.axis_index("c")
    c0 = wid * _CPW
    pltpu.sync_copy(idx_hbm.at[pl.ds(c0 * _P, _CPW * _P)], idx_v)

    ins, outs, isems, osems = (in0, in1), (out0, out1), (si0, si1), (so0, so1)
    nb = _B - _BT

    def cp_in(s, buf, sem):
        cl, b = s // nb, _BT + s % nb
        return pltpu.make_async_copy(x_hbm.at[b, c0 + cl], buf, sem)

    def cp_out(s, buf, sem):
        cl, b = s // nb, _BT + s % nb
        return pltpu.make_async_copy(buf, out_hbm.at[b, c0 + cl], sem)

    cp_in(0, ins[0], isems[0]).start()

    def pair_body(i, carry):
        s0 = i * 2
        for par in range(2):
            s = s0 + par
            nxt = s + 1
            npar = (par + 1) % 2

            @pl.when(nxt < _NSLAB)
            def _():
                cp_in(nxt, ins[npar], isems[npar]).start()

            cp_in(s, ins[par], isems[par]).wait()

            @pl.when(i > 0)
            def _():
                cp_out(s - 2, outs[par], osems[par]).wait()

            _gather_slab(ins[par], outs[par], idx_v, s // nb)
            cp_out(s, outs[par], osems[par]).start()
        return carry

    lax.fori_loop(0, _NSLAB // 2, pair_body, 0)
    cp_out(_NSLAB - 2, outs[0], osems[0]).wait()
    cp_out(_NSLAB - 1, outs[1], osems[1]).wait()


def _tc_body(prev_ref, x_ref, idx_ref, o_ref):
    x = x_ref[0, 0]
    idx2 = idx_ref[0]
    ih = idx2 & (_PH - 1)
    m = idx2 < _PH
    x0 = x[:, :_PH]
    x1 = x[:, _PH:]
    for k in range(2):
        ik = ih[:, k * _PH:(k + 1) * _PH]
        mk = m[:, k * _PH:(k + 1) * _PH]
        ib = jnp.broadcast_to(ik, (_E, _PH))
        g0 = jnp.take_along_axis(x0, ib, axis=1)
        g1 = jnp.take_along_axis(x1, ib, axis=1)
        mb = jnp.broadcast_to(mk, (_E, _PH))
        o_ref[0, 0, :, k * _PH:(k + 1) * _PH] = jnp.where(mb, g0, g1)


def _tc_fill(partial, X, idxmat):
    return pl.pallas_call(
        _tc_body,
        grid=(_BT, _C),
        in_specs=[
            pl.BlockSpec(memory_space=pl.ANY),
            pl.BlockSpec((1, 1, _E, _P), lambda b, c: (b, c, 0, 0)),
            pl.BlockSpec((1, 1, _P), lambda b, c: (c, 0, 0)),
        ],
        out_specs=pl.BlockSpec((1, 1, _E, _P), lambda b, c: (b, c, 0, 0)),
        out_shape=jax.ShapeDtypeStruct((_B, _C, _E, _P), jnp.float32),
        input_output_aliases={0: 0},
    )(partial, X, idxmat.reshape(_C, 1, _P))


@jax.jit
def _shuffle(X, idxmat):
    kern = pl.kernel(
        _shuffle_body,
        mesh=plsc.VectorSubcoreMesh(core_axis_name="c", subcore_axis_name="s"),
        compiler_params=pltpu.CompilerParams(needs_layout_passes=False),
        out_type=jax.ShapeDtypeStruct((_B, _C, _E, _P), jnp.float32),
        scratch_types=[
            pltpu.VMEM((_CPW * _P,), jnp.int32),
            pltpu.VMEM((_E, _P), jnp.float32),
            pltpu.VMEM((_E, _P), jnp.float32),
            pltpu.VMEM((_E, _P), jnp.float32),
            pltpu.VMEM((_E, _P), jnp.float32),
            pltpu.SemaphoreType.DMA,
            pltpu.SemaphoreType.DMA,
            pltpu.SemaphoreType.DMA,
            pltpu.SemaphoreType.DMA,
        ],
    )
    partial = kern(X, idxmat.reshape(_C * _P))
    return _tc_fill(partial, X, idxmat)


def kernel(X, perm_tensor, random_idx):
    idx = lax.dynamic_index_in_dim(perm_tensor, random_idx, 0, keepdims=False)
    return _shuffle(X, idx)


# speedup vs baseline: 5.4685x; 5.4685x over previous
import jax
import jax.numpy as jnp
from jax import lax
from jax.experimental import pallas as pl
from jax.experimental.pallas import tpu as pltpu
from jax.experimental.pallas import tpu_sc as plsc

_B, _C, _E, _P = 32, 64, 64, 256
_L = 16
_NC, _NS = 2, 16
_NW = _NC * _NS
_CPW = _C // _NW
_NSLAB = _B * _CPW


def _gather_slab(in_ref, out_ref, idx_v, cl):
    base = [idx_v[pl.ds(cl * _P + j * _L, _L)] for j in range(_P // _L)]

    @plsc.parallel_loop(0, _E, unroll=2)
    def e_body(e):
        e_vec = jnp.full((_L,), e, dtype=jnp.int32)
        for j in range(_P // _L):
            g = plsc.load_gather(in_ref, [e_vec, base[j]])
            out_ref[e, pl.ds(j * _L, _L)] = g


def _shuffle_body(x_hbm, idx_hbm, out_hbm, idx_v, in0, in1, out0, out1,
                  si0, si1, so0, so1):
    wid = lax.axis_index("s") * _NC + lax.axis_index("c")
    c0 = wid * _CPW
    pltpu.sync_copy(idx_hbm.at[pl.ds(c0 * _P, _CPW * _P)], idx_v)

    ins, outs, isems, osems = (in0, in1), (out0, out1), (si0, si1), (so0, so1)

    def cp_in(s, buf, sem):
        cl, b = s // _B, s % _B
        return pltpu.make_async_copy(x_hbm.at[b, c0 + cl], buf, sem)

    def cp_out(s, buf, sem):
        cl, b = s // _B, s % _B
        return pltpu.make_async_copy(buf, out_hbm.at[b, c0 + cl], sem)

    cp_in(0, ins[0], isems[0]).start()

    def pair_body(i, carry):
        s0 = i * 2
        for par in range(2):
            s = s0 + par
            nxt = s + 1
            npar = (par + 1) % 2

            @pl.when(nxt < _NSLAB)
            def _():
                cp_in(nxt, ins[npar], isems[npar]).start()

            cp_in(s, ins[par], isems[par]).wait()

            @pl.when(i > 0)
            def _():
                cp_out(s - 2, outs[par], osems[par]).wait()

            _gather_slab(ins[par], outs[par], idx_v, s // _B)
            cp_out(s, outs[par], osems[par]).start()
        return carry

    lax.fori_loop(0, _NSLAB // 2, pair_body, 0)
    cp_out(_NSLAB - 2, outs[0], osems[0]).wait()
    cp_out(_NSLAB - 1, outs[1], osems[1]).wait()


@jax.jit
def _shuffle(X, idxmat):
    kern = pl.kernel(
        _shuffle_body,
        mesh=plsc.VectorSubcoreMesh(core_axis_name="c", subcore_axis_name="s"),
        compiler_params=pltpu.CompilerParams(needs_layout_passes=False),
        out_type=jax.ShapeDtypeStruct((_B, _C, _E, _P), jnp.float32),
        scratch_types=[
            pltpu.VMEM((_CPW * _P,), jnp.int32),
            pltpu.VMEM((_E, _P), jnp.float32),
            pltpu.VMEM((_E, _P), jnp.float32),
            pltpu.VMEM((_E, _P), jnp.float32),
            pltpu.VMEM((_E, _P), jnp.float32),
            pltpu.SemaphoreType.DMA,
            pltpu.SemaphoreType.DMA,
            pltpu.SemaphoreType.DMA,
            pltpu.SemaphoreType.DMA,
        ],
    )
    return kern(X, idxmat.reshape(_C * _P))


def kernel(X, perm_tensor, random_idx):
    idx = lax.dynamic_index_in_dim(perm_tensor, random_idx, 0, keepdims=False)
    return _shuffle(X, idx)
